# baseline (speedup 1.0000x reference)
import functools

import jax
import jax.numpy as jnp
from jax import lax
from jax.experimental import pallas as pl
from jax.experimental.pallas import tpu as pltpu
from jax.experimental.pallas import tpu_sc as plsc

_NC = 2
_NS = 16
_NW = _NC * _NS
_NVOX = 2 * 32 * 64 * 64
_CHUNK = _NVOX // _NW
_ZB = 4


def _winner_code(x):
    rows = []
    for idv in range(1, 16):
        e = (x == idv).astype(jnp.bfloat16)
        zs = e[0] + e[1]
        rows.append(zs.reshape(1, 128, 128))
    zsum = jnp.concatenate(rows, axis=0)

    r = lax.broadcasted_iota(jnp.int32, (128, 64), 0)
    c = lax.broadcasted_iota(jnp.int32, (128, 64), 1)
    pair = ((r // 2) == c).astype(jnp.bfloat16)

    t = lax.dot_general(zsum, pair, (((1,), (0,)), ((), ())),
                        preferred_element_type=jnp.float32)
    u = lax.dot_general(t.astype(jnp.bfloat16), pair, (((1,), (0,)), ((), ())),
                        preferred_element_type=jnp.float32)

    c1 = jnp.zeros((64, 64), jnp.int32)
    c2 = jnp.zeros((64, 64), jnp.int32)
    for idv in range(1, 16):
        w = u[idv - 1] >= 3.5
        c2 = jnp.where(w & (c1 > 0) & (c2 == 0), idv, c2)
        c1 = jnp.where(w & (c1 == 0), idv, c1)
    return c1 | (c2 << 4)


def _codes_body(inst_ref, code_ref):
    x = inst_ref[0]
    for dz in range(_ZB):
        code_ref[0, dz] = _winner_code(x[2 * dz:2 * dz + 2])


def _sc_body(code_hbm, feat_hbm, part_hbm,
             codes_v, acti_v, actc_v, idx_v, vals_v, acc_v, sem):
    wid = lax.axis_index("s") * _NC + lax.axis_index("c")
    base = wid * _CHUNK

    zv = jnp.zeros((16,), jnp.float32)
    for seg in range(32):
        for g in range(5):
            acc_v[seg, pl.ds(g * 16, 16)] = zv

    pltpu.sync_copy(code_hbm.at[pl.ds(base, _CHUNK)], codes_v)

    lanes = lax.broadcasted_iota(jnp.int32, (16,), 0)

    def scan_body(i, cursor):
        v = codes_v[pl.ds(i * 16, 16)]
        m = v != 0
        npos = jnp.sum(m.astype(jnp.int32))

        @pl.when(npos > 0)
        def _store():
            idx = base + i * 16 + lanes
            plsc.store_compressed(actc_v.at[pl.ds(cursor, 16)], v, mask=m)
            plsc.store_compressed(acti_v.at[pl.ds(cursor, 16)], idx, mask=m)

        return cursor + npos

    n = lax.fori_loop(0, _CHUNK // 16, scan_body, 0)

    ones = jnp.ones((16,), jnp.float32)

    def group_body(g, carry):
        off = g * 16
        rem = n - off
        lm = lanes < rem
        vi = jnp.where(lm, acti_v[pl.ds(off, 16)], 0)
        va = jnp.where(lm, actc_v[pl.ds(off, 16)], 0)
        b = vi >> 17
        fbase = (b << 23) | (vi & 131071)
        id1 = va & 15
        id2 = (va >> 4) & 15
        seg1 = (b << 4) | id1
        seg2 = (b << 4) | id2
        lm2 = lm & (id2 > 0)

        for j in range(8):
            for cc in range(8):
                ch = 8 * j + cc
                idx_v[j, pl.ds(cc * 16, 16)] = fbase + (ch << 17)
        copies = [pltpu.async_copy(feat_hbm.at[idx_v.at[j]], vals_v.at[j], sem)
                  for j in range(8)]
        for cp in copies:
            cp.wait()
        for j in range(8):
            for cc in range(8):
                ch = 8 * j + cc
                cvec = jnp.full((16,), ch, jnp.int32)
                v = vals_v[j, pl.ds(cc * 16, 16)]
                plsc.addupdate_scatter(acc_v, [seg1, cvec], v, mask=lm)
                plsc.addupdate_scatter(acc_v, [seg2, cvec], v, mask=lm2)
        c64 = jnp.full((16,), 64, jnp.int32)
        plsc.addupdate_scatter(acc_v, [seg1, c64], ones, mask=lm)
        plsc.addupdate_scatter(acc_v, [seg2, c64], ones, mask=lm2)
        return carry

    ngroups = (n + 15) // 16
    lax.fori_loop(0, ngroups, group_body, 0)

    pltpu.sync_copy(acc_v, part_hbm.at[wid])


def _finalize_body(part_ref, w1_ref, w2_ref, w3_ref, b3_ref, out_ref):
    a = jnp.sum(part_ref[...], axis=0)
    sums = a[:, 0:64]
    cv = a[:, 64:65]
    emb = sums / jnp.where(cv > 0, cv, 1.0)
    h = lax.dot_general(emb, w1_ref[...], (((1,), (1,)), ((), ())),
                        preferred_element_type=jnp.float32)
    h = jnp.maximum(h, 0.0)
    h = lax.dot_general(h, w2_ref[...], (((1,), (1,)), ((), ())),
                        preferred_element_type=jnp.float32)
    h = jnp.maximum(h, 0.0)
    y = lax.dot_general(h, w3_ref[...], (((1,), (1,)), ((), ())),
                        preferred_element_type=jnp.float32) + b3_ref[...]
    out_ref[...] = jnp.where(cv > 0, y, jnp.nan)


def kernel(features, instances, W1, W2, W3, b3):
    B, C, Z, H, W = features.shape

    codes = pl.pallas_call(
        _codes_body,
        grid=(B, Z // _ZB),
        in_specs=[pl.BlockSpec((1, 2 * _ZB, 2 * H, 2 * W),
                               lambda b, z: (b, z, 0, 0))],
        out_specs=pl.BlockSpec((1, _ZB, H, W), lambda b, z: (b, z, 0, 0)),
        out_shape=jax.ShapeDtypeStruct((B, Z, H, W), jnp.int32),
    )(instances)

    codes_flat = codes.reshape(_NVOX)
    feat_flat = features.reshape(B * C * Z * H * W)

    mesh = plsc.VectorSubcoreMesh(core_axis_name="c", subcore_axis_name="s")
    sc_call = functools.partial(
        pl.kernel,
        mesh=mesh,
        compiler_params=pltpu.CompilerParams(needs_layout_passes=False),
        out_type=jax.ShapeDtypeStruct((_NW, 32, 80), jnp.float32),
        scratch_types=[
            pltpu.VMEM((_CHUNK,), jnp.int32),
            pltpu.VMEM((_CHUNK,), jnp.int32),
            pltpu.VMEM((_CHUNK,), jnp.int32),
            pltpu.VMEM((8, 128), jnp.int32),
            pltpu.VMEM((8, 128), jnp.float32),
            pltpu.VMEM((32, 80), jnp.float32),
            pltpu.SemaphoreType.DMA,
        ],
    )(_sc_body)
    partials = sc_call(codes_flat, feat_flat)

    y = pl.pallas_call(
        _finalize_body,
        in_specs=[
            pl.BlockSpec((_NW, 32, 80), lambda: (0, 0, 0)),
            pl.BlockSpec((64, 64), lambda: (0, 0)),
            pl.BlockSpec((64, 64), lambda: (0, 0)),
            pl.BlockSpec((32, 64), lambda: (0, 0)),
            pl.BlockSpec((1, 32), lambda: (0, 0)),
        ],
        out_specs=pl.BlockSpec((32, 32), lambda: (0, 0)),
        out_shape=jax.ShapeDtypeStruct((32, 32), jnp.float32),
    )(partials, W1, W2, W3, b3.reshape(1, 32))

    return jnp.concatenate([y[1:16], y[17:32]], axis=0)


# speedup vs baseline: 1.0283x; 1.0283x over previous
import functools

import jax
import jax.numpy as jnp
from jax import lax
from jax.experimental import pallas as pl
from jax.experimental.pallas import tpu as pltpu
from jax.experimental.pallas import tpu_sc as plsc

_NC = 2
_NS = 16
_NW = _NC * _NS
_NVOX = 2 * 32 * 64 * 64
_CHUNK = _NVOX // _NW
_ZB = 4


def _winner_code(x):
    rows = []
    for idv in range(1, 16):
        e = (x == idv).astype(jnp.bfloat16)
        zs = e[0] + e[1]
        rows.append(zs.reshape(1, 128, 128))
    zsum = jnp.concatenate(rows, axis=0)

    r = lax.broadcasted_iota(jnp.int32, (128, 64), 0)
    c = lax.broadcasted_iota(jnp.int32, (128, 64), 1)
    pair = ((r // 2) == c).astype(jnp.bfloat16)

    t = lax.dot_general(zsum, pair, (((1,), (0,)), ((), ())),
                        preferred_element_type=jnp.float32)
    u = lax.dot_general(t.astype(jnp.bfloat16), pair, (((1,), (0,)), ((), ())),
                        preferred_element_type=jnp.float32)

    c1 = jnp.zeros((64, 64), jnp.int32)
    c2 = jnp.zeros((64, 64), jnp.int32)
    for idv in range(1, 16):
        w = u[idv - 1] >= 3.5
        c2 = jnp.where(w & (c1 > 0) & (c2 == 0), idv, c2)
        c1 = jnp.where(w & (c1 == 0), idv, c1)
    return c1 | (c2 << 4)


def _codes_body(inst_ref, code_ref):
    x = inst_ref[0]
    for dz in range(_ZB):
        code_ref[0, dz] = _winner_code(x[2 * dz:2 * dz + 2])


def _sc_body(code_hbm, feat_hbm, part_hbm,
             codes_v, acti_v, actc_v, idx_v, vals_v, acc_v, sem):
    wid = lax.axis_index("s") * _NC + lax.axis_index("c")
    base = wid * _CHUNK

    zv = jnp.zeros((16,), jnp.float32)
    for seg in range(32):
        for g in range(5):
            acc_v[seg, pl.ds(g * 16, 16)] = zv

    pltpu.sync_copy(code_hbm.at[pl.ds(base, _CHUNK)], codes_v)

    lanes = lax.broadcasted_iota(jnp.int32, (16,), 0)

    def scan_body(i, cursor):
        v = codes_v[pl.ds(i * 16, 16)]
        m = v != 0
        npos = jnp.sum(m.astype(jnp.int32))
        idx = base + i * 16 + lanes
        plsc.store_compressed(actc_v.at[pl.ds(cursor, 16)], v, mask=m)
        plsc.store_compressed(acti_v.at[pl.ds(cursor, 16)], idx, mask=m)
        return cursor + npos

    n = lax.fori_loop(0, _CHUNK // 16, scan_body, 0)

    ones = jnp.ones((16,), jnp.float32)

    def group_body(g, carry):
        off = g * 16
        rem = n - off
        lm = lanes < rem
        vi = jnp.where(lm, acti_v[pl.ds(off, 16)], 0)
        va = jnp.where(lm, actc_v[pl.ds(off, 16)], 0)
        b = vi >> 17
        fbase = (b << 23) | (vi & 131071)
        id1 = va & 15
        id2 = (va >> 4) & 15
        seg1 = (b << 4) | id1
        seg2 = (b << 4) | id2
        lm2 = lm & (id2 > 0)

        for j in range(8):
            for cc in range(8):
                ch = 8 * j + cc
                idx_v[j, pl.ds(cc * 16, 16)] = fbase + (ch << 17)
        copies = [pltpu.async_copy(feat_hbm.at[idx_v.at[j]], vals_v.at[j], sem)
                  for j in range(8)]
        for cp in copies:
            cp.wait()
        for j in range(8):
            for cc in range(8):
                ch = 8 * j + cc
                cvec = jnp.full((16,), ch, jnp.int32)
                v = vals_v[j, pl.ds(cc * 16, 16)]
                plsc.addupdate_scatter(acc_v, [seg1, cvec], v, mask=lm)
                plsc.addupdate_scatter(acc_v, [seg2, cvec], v, mask=lm2)
        c64 = jnp.full((16,), 64, jnp.int32)
        plsc.addupdate_scatter(acc_v, [seg1, c64], ones, mask=lm)
        plsc.addupdate_scatter(acc_v, [seg2, c64], ones, mask=lm2)
        return carry

    ngroups = (n + 15) // 16
    lax.fori_loop(0, ngroups, group_body, 0)

    pltpu.sync_copy(acc_v, part_hbm.at[wid])


def _finalize_body(part_ref, w1_ref, w2_ref, w3_ref, b3_ref, out_ref):
    a = jnp.sum(part_ref[...], axis=0)
    sums = a[:, 0:64]
    cv = a[:, 64:65]
    emb = sums / jnp.where(cv > 0, cv, 1.0)
    h = lax.dot_general(emb, w1_ref[...], (((1,), (1,)), ((), ())),
                        preferred_element_type=jnp.float32)
    h = jnp.maximum(h, 0.0)
    h = lax.dot_general(h, w2_ref[...], (((1,), (1,)), ((), ())),
                        preferred_element_type=jnp.float32)
    h = jnp.maximum(h, 0.0)
    y = lax.dot_general(h, w3_ref[...], (((1,), (1,)), ((), ())),
                        preferred_element_type=jnp.float32) + b3_ref[...]
    out_ref[...] = jnp.where(cv > 0, y, jnp.nan)


def kernel(features, instances, W1, W2, W3, b3):
    B, C, Z, H, W = features.shape

    codes = pl.pallas_call(
        _codes_body,
        grid=(B, Z // _ZB),
        in_specs=[pl.BlockSpec((1, 2 * _ZB, 2 * H, 2 * W),
                               lambda b, z: (b, z, 0, 0))],
        out_specs=pl.BlockSpec((1, _ZB, H, W), lambda b, z: (b, z, 0, 0)),
        out_shape=jax.ShapeDtypeStruct((B, Z, H, W), jnp.int32),
    )(instances)

    codes_flat = codes.reshape(_NVOX)
    feat_flat = features.reshape(B * C * Z * H * W)

    mesh = plsc.VectorSubcoreMesh(core_axis_name="c", subcore_axis_name="s")
    sc_call = functools.partial(
        pl.kernel,
        mesh=mesh,
        compiler_params=pltpu.CompilerParams(needs_layout_passes=False),
        out_type=jax.ShapeDtypeStruct((_NW, 32, 80), jnp.float32),
        scratch_types=[
            pltpu.VMEM((_CHUNK,), jnp.int32),
            pltpu.VMEM((_CHUNK,), jnp.int32),
            pltpu.VMEM((_CHUNK,), jnp.int32),
            pltpu.VMEM((8, 128), jnp.int32),
            pltpu.VMEM((8, 128), jnp.float32),
            pltpu.VMEM((32, 80), jnp.float32),
            pltpu.SemaphoreType.DMA,
        ],
    )(_sc_body)
    partials = sc_call(codes_flat, feat_flat)

    y = pl.pallas_call(
        _finalize_body,
        in_specs=[
            pl.BlockSpec((_NW, 32, 80), lambda: (0, 0, 0)),
            pl.BlockSpec((64, 64), lambda: (0, 0)),
            pl.BlockSpec((64, 64), lambda: (0, 0)),
            pl.BlockSpec((32, 64), lambda: (0, 0)),
            pl.BlockSpec((1, 32), lambda: (0, 0)),
        ],
        out_specs=pl.BlockSpec((32, 32), lambda: (0, 0)),
        out_shape=jax.ShapeDtypeStruct((32, 32), jnp.float32),
    )(partials, W1, W2, W3, b3.reshape(1, 32))

    return jnp.concatenate([y[1:16], y[17:32]], axis=0)
